# Initial kernel scaffold; baseline (speedup 1.0000x reference)
import jax
import jax.numpy as jnp
from jax.experimental import pallas as pl


def kernel(y_pred, y_true, points, point_labels):
    raise NotImplementedError("write your pallas kernel here")


# speedup vs baseline: 7.1144x; 7.1144x over previous
import functools
import math

import jax
import jax.numpy as jnp
from jax import lax
from jax.experimental import pallas as pl
from jax.experimental.pallas import tpu as pltpu
from jax.experimental.pallas import tpu_sc as plsc

B, NCLS, H, W = 8, 3, 512, 512
L = 200
RADIUS = 15
SIGMA = RADIUS // 3
C_CONST = float(math.log(math.e + 2.0))
INV_2SIG2 = 1.0 / (2.0 * SIGMA * SIGMA)
NROWS = RADIUS
HALF = RADIUS // 2
CHW = W // 16
ROWCH = H * CHW
ED = [math.exp(-((d - HALF) ** 2) * INV_2SIG2) for d in range(NROWS)]


def _sc_body(NC, NW, PPW, NBLK,
             ytrue_hbm, ypred_hbm, pts_hbm, out_hbm,
             pts_v, idx_t, idx_p, vals_t, vals_p,
             acc_v, sem):
    wid = lax.axis_index("s") * NC + lax.axis_index("c")
    base = wid * PPW
    pltpu.sync_copy(pts_hbm.at[pl.ds(base, PPW)], pts_v)

    lane = lax.iota(jnp.int32, 16)
    acc = jnp.zeros((16,), jnp.float32)

    for blk in range(NBLK):
        def build(p, carry):
            sp = blk * 16 + p
            v = pts_v[sp, :]
            i_p = v[0]
            j_p = v[1]
            b_p = v[3]
            ii_c = jnp.clip(i_p + lane - HALF, 0, H - 1)
            c0 = lax.shift_right_arithmetic(j_p - HALF, 4)
            rowbase = b_p * ROWCH + ii_c * CHW
            poff = b_p * 2 * ROWCH
            for s in range(2):
                cc = jnp.clip(c0 + s, 0, CHW - 1)
                idxv = rowbase + cc
                off = p * 32 + s * 16
                idx_t[pl.ds(off, 16)] = idxv
                for c in range(NCLS):
                    idx_p[pl.ds(c * 512 + off, 16)] = idxv + poff + c * ROWCH
            return carry

        lax.fori_loop(0, 16, build, 0)

        copies = []
        for g in range(4):
            copies.append(pltpu.async_copy(
                ytrue_hbm.at[idx_t.at[pl.ds(g * 128, 128)]],
                vals_t.at[pl.ds(g * 128, 128)], sem))
        for g in range(12):
            copies.append(pltpu.async_copy(
                ypred_hbm.at[idx_p.at[pl.ds(g * 128, 128)]],
                vals_p.at[pl.ds(g * 128, 128)], sem))
        for cp in copies:
            cp.wait()

        def comp(p, acc_in):
            sp = blk * 16 + p
            v = pts_v[sp, :]
            i_p = v[0]
            j_p = v[1]
            lab = v[2]
            c0 = lax.shift_right_arithmetic(j_p - HALF, 4)
            a = acc_in
            for s in range(2):
                col = (c0 + s) * 16 + lane
                dj = col - j_p
                djf = dj.astype(jnp.float32)
                colok = (dj >= -HALF) & (dj <= HALF - 1) & (col >= 0) & (col < W)
                colw = jnp.where(colok, jnp.exp(djf * djf * (-INV_2SIG2)), 0.0)
                for d in range(NROWS):
                    ii = i_p + (d - HALF)
                    rw = jnp.where((ii >= 0) & (ii < H), ED[d], 0.0)
                    row = p * 32 + s * 16 + d
                    t16 = vals_t[row, :]
                    x0 = vals_p[row, :]
                    x1 = vals_p[512 + row, :]
                    x2 = vals_p[1024 + row, :]
                    m = jnp.maximum(x0, jnp.maximum(x1, x2))
                    e0 = jnp.exp(x0 - m)
                    e1 = jnp.exp(x1 - m)
                    e2 = jnp.exp(x2 - m)
                    pt = jnp.where(t16 == 0, e0, jnp.where(t16 == 1, e1, e2)) \
                        / (e0 + e1 + e2)
                    wgt = jnp.where(t16 == lab, colw * rw, 0.0)
                    a = a + wgt * (C_CONST - pt)
            return a

        acc = lax.fori_loop(0, 16, comp, acc)

    acc_v[...] = acc
    pltpu.sync_copy(acc_v, out_hbm.at[wid])


def kernel(y_pred, y_true, points, point_labels):
    info = plsc.get_sparse_core_info()
    NC, NS = info.num_cores, info.num_subcores
    NW = NC * NS
    nblk = -(-(B * L) // (NW * 16))
    PPW = nblk * 16
    NPTS = PPW * NW

    ytrue2 = y_true.reshape(B * H * W // 16, 16)
    ypred2 = y_pred.reshape(B * NCLS * H * W // 16, 16)

    i_all = points[:, :, 0].reshape(-1)
    j_all = points[:, :, 1].reshape(-1)
    l_all = point_labels[:, :, 0].reshape(-1)
    b_all = jnp.repeat(jnp.arange(B, dtype=jnp.int32), L)
    pad = NPTS - B * L

    def prep(x, fill):
        x = jnp.concatenate([x, jnp.full((pad,), fill, jnp.int32)])
        return x.reshape(PPW, NW).T.reshape(-1)

    pts_packed = jnp.stack(
        [prep(i_all, 0), prep(j_all, 0), prep(l_all, -1), prep(b_all, 0)]
        + [jnp.zeros((NPTS,), jnp.int32)] * 12, axis=1)

    mesh = plsc.VectorSubcoreMesh(core_axis_name="c", subcore_axis_name="s")
    f = pl.kernel(
        functools.partial(_sc_body, NC, NW, PPW, nblk),
        out_type=jax.ShapeDtypeStruct((NW, 16), jnp.float32),
        mesh=mesh,
        compiler_params=pltpu.CompilerParams(use_tc_tiling_on_sc=False),
        scratch_types=[
            pltpu.VMEM((PPW, 16), jnp.int32),
            pltpu.VMEM((512,), jnp.int32),
            pltpu.VMEM((1536,), jnp.int32),
            pltpu.VMEM((512, 16), jnp.int32),
            pltpu.VMEM((1536, 16), jnp.float32),
            pltpu.VMEM((16,), jnp.float32),
            pltpu.SemaphoreType.DMA,
        ],
    )
    out = f(ytrue2, ypred2, pts_packed)
    return jnp.sum(out) * (1.0 / (B * H * W))
